# baseline (speedup 1.0000x reference)
import functools

import jax
import jax.numpy as jnp
from jax import lax
from jax.experimental import pallas as pl
from jax.experimental.pallas import tpu as pltpu
from jax.experimental.pallas import tpu_sc as plsc

_NUM_CORES = 2
_NUM_SUBCORES = 16
_NUM_WORKERS = _NUM_CORES * _NUM_SUBCORES
_L = 16

_SC_ROWS = 2048
_TC_BB = 512


@functools.lru_cache(maxsize=None)
def _make_sc(V, D, B_sc, nch):
    b_per_w = B_sc // _NUM_WORKERS
    rows_c = b_per_w // nch
    mesh = plsc.VectorSubcoreMesh(core_axis_name="c", subcore_axis_name="s")

    @functools.partial(
        pl.kernel,
        mesh=mesh,
        out_type=jax.ShapeDtypeStruct((B_sc, D), jnp.float32),
        scratch_types=[
            pltpu.MemorySpace.HBM((_NUM_WORKERS * 8, D), jnp.float32),
            pltpu.VMEM((8, D), jnp.float32),
            pltpu.VMEM((b_per_w,), jnp.int32),
            pltpu.VMEM((rows_c, D), jnp.float32),
            pltpu.VMEM((rows_c, D), jnp.float32),
            pltpu.SemaphoreType.DMA,
            pltpu.SemaphoreType.DMA,
            pltpu.SemaphoreType.DMA,
            pltpu.SemaphoreType.DMA,
        ],
    )
    def k(table_hbm, idx_hbm, out_hbm, priv_hbm, table_v, idx_v,
          buf0, buf1, sg0, sg1, sw0, sw1):
        wid = lax.axis_index("s") * _NUM_CORES + lax.axis_index("c")
        base = wid * b_per_w
        bufs = (buf0, buf1)
        sg = (sg0, sg1)
        sw = (sw0, sw1)
        pltpu.sync_copy(table_hbm, table_v.at[pl.ds(0, V)])
        pltpu.sync_copy(table_v, priv_hbm.at[pl.ds(wid * 8, 8)])
        pltpu.sync_copy(idx_hbm.at[pl.ds(base, b_per_w)], idx_v)
        for i in range(b_per_w // _L):
            sl = pl.ds(i * _L, _L)
            idx_v[sl] = idx_v[sl] + wid * 8
        writes = [None, None]
        for c in range(nch):
            b = c & 1
            if writes[b] is not None:
                writes[b].wait()
            idx_c = idx_v.at[pl.ds(c * rows_c, rows_c)]
            pltpu.async_copy(priv_hbm.at[idx_c], bufs[b], sg[b]).wait()
            writes[b] = pltpu.async_copy(
                bufs[b], out_hbm.at[pl.ds(base + c * rows_c, rows_c)], sw[b])
        for w in writes:
            if w is not None:
                w.wait()

    return k


def _onehot_body(idx_ref, table_ref, o_ref):
    idxb = idx_ref[0, 0, :]
    bb = idxb.shape[0]
    v = table_ref.shape[0]
    iota = lax.broadcasted_iota(jnp.int32, (bb, v), 1)
    onehot = (idxb[:, None] == iota).astype(jnp.float32)
    o_ref[...] = lax.dot_general(
        onehot, table_ref[...], (((1,), (0,)), ((), ())),
        preferred_element_type=jnp.float32)


@functools.lru_cache(maxsize=None)
def _make_tc(V, D, B_tc, BB):
    return pl.pallas_call(
        _onehot_body,
        grid=(B_tc // BB,),
        in_specs=[
            pl.BlockSpec((1, 1, BB), lambda i: (i, 0, 0)),
            pl.BlockSpec((V, D), lambda i: (0, 0)),
        ],
        out_specs=pl.BlockSpec((BB, D), lambda i: (i, 0)),
        out_shape=jax.ShapeDtypeStruct((B_tc, D), jnp.float32),
    )


def kernel(disease_embeds, disease_indices):
    V, D = disease_embeds.shape
    (B,) = disease_indices.shape
    idx = disease_indices.astype(jnp.int32)
    b_sc = _SC_ROWS
    b_tc = B - b_sc
    idx3 = idx[:b_tc].reshape(b_tc // _TC_BB, 1, _TC_BB)
    out_tc = _make_tc(V, D, b_tc, _TC_BB)(idx3, disease_embeds)
    nch = max(1, min(4, (b_sc // _NUM_WORKERS) // 32))
    out_sc = _make_sc(V, D, b_sc, nch)(disease_embeds, idx[b_tc:])
    return jnp.concatenate([out_tc, out_sc], axis=0)


# speedup vs baseline: 1.0005x; 1.0005x over previous
import functools

import jax
import jax.numpy as jnp
from jax import lax
from jax.experimental import pallas as pl
from jax.experimental.pallas import tpu as pltpu
from jax.experimental.pallas import tpu_sc as plsc

_NUM_CORES = 2
_NUM_SUBCORES = 16
_NUM_WORKERS = _NUM_CORES * _NUM_SUBCORES
_L = 16

_SC_ROWS = 2048
_TC_BB = 512


@functools.lru_cache(maxsize=None)
def _make_sc(V, D, B_sc, nch):
    b_per_w = B_sc // _NUM_WORKERS
    rows_c = b_per_w // nch
    mesh = plsc.VectorSubcoreMesh(core_axis_name="c", subcore_axis_name="s")

    @functools.partial(
        pl.kernel,
        mesh=mesh,
        out_type=jax.ShapeDtypeStruct((B_sc, D), jnp.float32),
        scratch_types=[
            pltpu.MemorySpace.HBM((_NUM_WORKERS * 8, D), jnp.float32),
            pltpu.VMEM((8, D), jnp.float32),
            pltpu.VMEM((b_per_w,), jnp.int32),
            pltpu.VMEM((rows_c, D), jnp.float32),
            pltpu.VMEM((rows_c, D), jnp.float32),
            pltpu.SemaphoreType.DMA,
            pltpu.SemaphoreType.DMA,
            pltpu.SemaphoreType.DMA,
            pltpu.SemaphoreType.DMA,
        ],
    )
    def k(table_hbm, idx_hbm, out_hbm, priv_hbm, table_v, idx_v,
          buf0, buf1, sg0, sg1, sw0, sw1):
        wid = lax.axis_index("s") * _NUM_CORES + lax.axis_index("c")
        base = wid * b_per_w
        bufs = (buf0, buf1)
        sg = (sg0, sg1)
        sw = (sw0, sw1)
        pltpu.sync_copy(table_hbm, table_v.at[pl.ds(0, V)])
        pltpu.sync_copy(table_v, priv_hbm.at[pl.ds(wid * 8, 8)])
        pltpu.sync_copy(idx_hbm.at[pl.ds(base, b_per_w)], idx_v)
        for i in range(b_per_w // _L):
            sl = pl.ds(i * _L, _L)
            idx_v[sl] = idx_v[sl] + wid * 8
        writes = [None, None]
        for c in range(nch):
            b = c & 1
            if writes[b] is not None:
                writes[b].wait()
            idx_c = idx_v.at[pl.ds(c * rows_c, rows_c)]
            pltpu.async_copy(priv_hbm.at[idx_c], bufs[b], sg[b]).wait()
            writes[b] = pltpu.async_copy(
                bufs[b], out_hbm.at[pl.ds(base + c * rows_c, rows_c)], sw[b])
        for w in writes:
            if w is not None:
                w.wait()

    return k


def _onehot_body(idx_ref, table_ref, o_ref):
    idxb = idx_ref[0, 0, :]
    bb = idxb.shape[0]
    v = table_ref.shape[0]
    iota = lax.broadcasted_iota(jnp.int32, (bb, v), 1)
    onehot = (idxb[:, None] == iota).astype(jnp.float32)
    o_ref[...] = lax.dot_general(
        onehot, table_ref[...], (((1,), (0,)), ((), ())),
        preferred_element_type=jnp.float32)


@functools.lru_cache(maxsize=None)
def _make_tc(V, D, B_tc, BB):
    return pl.pallas_call(
        _onehot_body,
        grid=(B_tc // BB,),
        in_specs=[
            pl.BlockSpec((1, 1, BB), lambda i: (i, 0, 0)),
            pl.BlockSpec((V, D), lambda i: (0, 0)),
        ],
        out_specs=pl.BlockSpec((BB, D), lambda i: (i, 0)),
        out_shape=jax.ShapeDtypeStruct((B_tc, D), jnp.float32),
    )


def kernel(disease_embeds, disease_indices):
    V, D = disease_embeds.shape
    (B,) = disease_indices.shape
    idx = disease_indices.astype(jnp.int32)
    b_sc = _SC_ROWS
    b_tc = B - b_sc
    idx3 = idx[:b_tc].reshape(b_tc // _TC_BB, 1, _TC_BB)
    nch = max(1, min(4, (b_sc // _NUM_WORKERS) // 32))
    out_sc = _make_sc(V, D, b_sc, nch)(disease_embeds, idx[b_tc:])
    out_tc = _make_tc(V, D, b_tc, _TC_BB)(idx3, disease_embeds)
    return jnp.concatenate([out_tc, out_sc], axis=0)


# speedup vs baseline: 1.1493x; 1.1487x over previous
import functools

import jax
import jax.numpy as jnp
from jax import lax
from jax.experimental import pallas as pl
from jax.experimental.pallas import tpu as pltpu
from jax.experimental.pallas import tpu_sc as plsc

_NUM_CORES = 2
_NUM_SUBCORES = 16
_NUM_WORKERS = _NUM_CORES * _NUM_SUBCORES
_L = 16
_NCH = 4
_RB = 4


@functools.lru_cache(maxsize=None)
def _make_sc(V, D, B):
    assert B % (_NUM_WORKERS * _NCH) == 0 and D % _L == 0
    b_per_w = B // _NUM_WORKERS
    rows_c = b_per_w // _NCH
    dch = D // _L
    mesh = plsc.VectorSubcoreMesh(core_axis_name="c", subcore_axis_name="s")

    @functools.partial(
        pl.kernel,
        mesh=mesh,
        out_type=jax.ShapeDtypeStruct((B, D), jnp.float32),
        scratch_types=[
            pltpu.VMEM((V, D), jnp.float32),
            pltpu.VMEM((b_per_w, _L), jnp.int32),
            pltpu.VMEM((rows_c, D), jnp.float32),
            pltpu.VMEM((rows_c, D), jnp.float32),
            pltpu.SemaphoreType.DMA,
            pltpu.SemaphoreType.DMA,
        ],
    )
    def k(table_hbm, idxrep_hbm, out_hbm, tab_v, idxr_v, buf0, buf1,
          sw0, sw1):
        wid = lax.axis_index("s") * _NUM_CORES + lax.axis_index("c")
        base = wid * b_per_w
        bufs = (buf0, buf1)
        sw = (sw0, sw1)
        pltpu.sync_copy(table_hbm, tab_v)
        pltpu.sync_copy(idxrep_hbm.at[pl.ds(base, b_per_w)], idxr_v)

        writes = [None, None]
        for c in range(_NCH):
            b = c & 1
            if writes[b] is not None:
                writes[b].wait()
            buf = bufs[b]
            for rb in range(rows_c // _RB):
                r0 = rb * _RB
                oh = []
                for j in range(_RB):
                    rv = idxr_v[c * rows_c + r0 + j, :]
                    oh.append([rv == v for v in range(V - 1)])

                def body(i, carry, oh=oh, buf=buf, r0=r0):
                    sl = pl.ds(i * _L, _L)
                    t = [tab_v[v, sl] for v in range(V)]
                    for j in range(_RB):
                        col = t[V - 1]
                        for v in range(V - 2, -1, -1):
                            col = jnp.where(oh[j][v], t[v], col)
                        buf[r0 + j, sl] = col
                    return carry

                lax.fori_loop(0, dch, body, jnp.int32(0))
            writes[b] = pltpu.async_copy(
                buf, out_hbm.at[pl.ds(base + c * rows_c, rows_c)], sw[b])
        for w in writes:
            if w is not None:
                w.wait()

    return k


def kernel(disease_embeds, disease_indices):
    V, D = disease_embeds.shape
    (B,) = disease_indices.shape
    idx_rep = jnp.broadcast_to(
        disease_indices.astype(jnp.int32)[:, None], (B, _L))
    return _make_sc(V, D, B)(disease_embeds, idx_rep)
